# Initial kernel scaffold; baseline (speedup 1.0000x reference)
import jax
import jax.numpy as jnp
from jax.experimental import pallas as pl


def kernel(x, w1, g1, b1, w2, g2, b2):
    raise NotImplementedError("write your pallas kernel here")


# speedup vs baseline: 1.6953x; 1.6953x over previous
import jax
import jax.numpy as jnp
from jax.experimental import pallas as pl
from jax.experimental.pallas import tpu as pltpu

_EPS = 1e-5
_VMEM = 64 * 1024 * 1024


def _colmask(L, Wp, W, dtype):
    col = jax.lax.broadcasted_iota(jnp.int32, (L, 1), 0) % Wp
    return (col < W).astype(dtype)


def _conv_stats_call(xpf, w, *, N, H, W, Cin, Cout):
    Wp = W + 2
    L = H * Wp
    P = (H + 3) * Wp

    def body(x_ref, w_ref, o_ref, s1_ref, s2_ref):
        acc = jnp.zeros((L, Cout), jnp.float32)
        for ky in range(3):
            for kx in range(3):
                t = ky * 3 + kx
                acc = acc + jnp.dot(
                    x_ref[0, pl.ds(ky * Wp + kx, L), :], w_ref[t],
                    preferred_element_type=jnp.float32)
        av = acc * _colmask(L, Wp, W, jnp.float32)
        s1_ref[0] = jnp.sum(av, axis=0, keepdims=True)
        s2_ref[0] = jnp.sum(av * av, axis=0, keepdims=True)
        o_ref[0] = acc.astype(jnp.bfloat16)

    return pl.pallas_call(
        body,
        out_shape=(
            jax.ShapeDtypeStruct((N, L, Cout), jnp.bfloat16),
            jax.ShapeDtypeStruct((N, 1, Cout), jnp.float32),
            jax.ShapeDtypeStruct((N, 1, Cout), jnp.float32),
        ),
        grid=(N,),
        in_specs=[
            pl.BlockSpec((1, P, Cin), lambda n: (n, 0, 0)),
            pl.BlockSpec((9, Cin, Cout), lambda n: (0, 0, 0)),
        ],
        out_specs=(
            pl.BlockSpec((1, L, Cout), lambda n: (n, 0, 0)),
            pl.BlockSpec((1, 1, Cout), lambda n: (n, 0, 0)),
            pl.BlockSpec((1, 1, Cout), lambda n: (n, 0, 0)),
        ),
        compiler_params=pltpu.CompilerParams(
            dimension_semantics=("parallel",),
            vmem_limit_bytes=_VMEM,
        ),
    )(xpf, w)


def _bn_conv_stats_call(c1, w, s1, s2, g, b, *, N, H, W, Cin, Cout):
    Wp = W + 2
    L = H * Wp
    P = (H + 3) * Wp
    m = float(N * H * W)

    def body(x_ref, w_ref, s1_ref, s2_ref, g_ref, b_ref,
             o_ref, t1_ref, t2_ref, buf):
        mean = jnp.sum(s1_ref[...], axis=0) / m
        var = jnp.maximum(jnp.sum(s2_ref[...], axis=0) / m - mean * mean, 0.0)
        scale = g_ref[...] * jax.lax.rsqrt(var + _EPS)
        shift = b_ref[...] - mean * scale

        x = x_ref[0].astype(jnp.float32)
        y = jnp.maximum(x * scale + shift, 0.0) * _colmask(L, Wp, W, jnp.float32)
        buf[...] = jnp.zeros((P, Cin), jnp.bfloat16)
        buf[pl.ds(Wp + 1, L), :] = y.astype(jnp.bfloat16)

        acc = jnp.zeros((L, Cout), jnp.float32)
        for ky in range(3):
            for kx in range(3):
                t = ky * 3 + kx
                acc = acc + jnp.dot(
                    buf[pl.ds(ky * Wp + kx, L), :], w_ref[t],
                    preferred_element_type=jnp.float32)
        av = acc * _colmask(L, Wp, W, jnp.float32)
        t1_ref[0] = jnp.sum(av, axis=0, keepdims=True)
        t2_ref[0] = jnp.sum(av * av, axis=0, keepdims=True)
        o_ref[0] = acc.astype(jnp.bfloat16)

    return pl.pallas_call(
        body,
        out_shape=(
            jax.ShapeDtypeStruct((N, L, Cout), jnp.bfloat16),
            jax.ShapeDtypeStruct((N, 1, Cout), jnp.float32),
            jax.ShapeDtypeStruct((N, 1, Cout), jnp.float32),
        ),
        grid=(N,),
        in_specs=[
            pl.BlockSpec((1, L, Cin), lambda n: (n, 0, 0)),
            pl.BlockSpec((9, Cin, Cout), lambda n: (0, 0, 0)),
            pl.BlockSpec((N, 1, Cin), lambda n: (0, 0, 0)),
            pl.BlockSpec((N, 1, Cin), lambda n: (0, 0, 0)),
            pl.BlockSpec((1, Cin), lambda n: (0, 0)),
            pl.BlockSpec((1, Cin), lambda n: (0, 0)),
        ],
        out_specs=(
            pl.BlockSpec((1, L, Cout), lambda n: (n, 0, 0)),
            pl.BlockSpec((1, 1, Cout), lambda n: (n, 0, 0)),
            pl.BlockSpec((1, 1, Cout), lambda n: (n, 0, 0)),
        ),
        scratch_shapes=[pltpu.VMEM((P, Cin), jnp.bfloat16)],
        compiler_params=pltpu.CompilerParams(
            dimension_semantics=("parallel",),
            vmem_limit_bytes=_VMEM,
        ),
    )(c1, w, s1, s2, g, b)


def _bn_relu_out_call(c2, s1, s2, g, b, *, N, H, W, Cout):
    Wp = W + 2
    L = H * Wp
    m = float(N * H * W)

    def body(x_ref, s1_ref, s2_ref, g_ref, b_ref, o_ref):
        mean = jnp.sum(s1_ref[...], axis=0) / m
        var = jnp.maximum(jnp.sum(s2_ref[...], axis=0) / m - mean * mean, 0.0)
        scale = g_ref[...] * jax.lax.rsqrt(var + _EPS)
        shift = b_ref[...] - mean * scale
        x = x_ref[0].astype(jnp.float32)
        o_ref[0] = jnp.maximum(x * scale + shift, 0.0)

    return pl.pallas_call(
        body,
        out_shape=jax.ShapeDtypeStruct((N, L, Cout), jnp.float32),
        grid=(N,),
        in_specs=[
            pl.BlockSpec((1, L, Cout), lambda n: (n, 0, 0)),
            pl.BlockSpec((N, 1, Cout), lambda n: (0, 0, 0)),
            pl.BlockSpec((N, 1, Cout), lambda n: (0, 0, 0)),
            pl.BlockSpec((1, Cout), lambda n: (0, 0)),
            pl.BlockSpec((1, Cout), lambda n: (0, 0)),
        ],
        out_specs=pl.BlockSpec((1, L, Cout), lambda n: (n, 0, 0)),
        compiler_params=pltpu.CompilerParams(
            dimension_semantics=("parallel",),
            vmem_limit_bytes=_VMEM,
        ),
    )(c2, s1, s2, g, b)


def kernel(x, w1, g1, b1, w2, g2, b2):
    N, C0, H0, W0 = x.shape
    H, W = H0 // 2, W0 // 2
    Wp = W + 2
    C1 = w1.shape[2]
    C2 = w2.shape[2]

    pooled = jnp.max(x.reshape(N, C0, H, 2, W, 2), axis=(3, 5))
    xh = jnp.transpose(pooled, (0, 2, 3, 1))
    xp = jnp.pad(xh, ((0, 0), (1, 2), (1, 1), (0, 0))).astype(jnp.bfloat16)
    xpf = xp.reshape(N, (H + 3) * Wp, C0)

    w1b = w1.astype(jnp.bfloat16)
    w2b = w2.astype(jnp.bfloat16)
    g1r, b1r = g1.reshape(1, C1), b1.reshape(1, C1)
    g2r, b2r = g2.reshape(1, C2), b2.reshape(1, C2)

    c1, s1, s2 = _conv_stats_call(xpf, w1b, N=N, H=H, W=W, Cin=C0, Cout=C1)
    c2, t1, t2 = _bn_conv_stats_call(c1, w2b, s1, s2, g1r, b1r,
                                     N=N, H=H, W=W, Cin=C1, Cout=C2)
    y = _bn_relu_out_call(c2, t1, t2, g2r, b2r, N=N, H=H, W=W, Cout=C2)

    yv = y.reshape(N, H, Wp, C2)[:, :, :W, :]
    return jnp.transpose(yv, (0, 3, 1, 2))


# speedup vs baseline: 2.2494x; 1.3268x over previous
import jax
import jax.numpy as jnp
from jax.experimental import pallas as pl
from jax.experimental.pallas import tpu as pltpu

_EPS = 1e-5
_VMEM = 64 * 1024 * 1024


def _edge_masks(L, W, dtype):
    col = jax.lax.broadcasted_iota(jnp.int32, (1, L), 1) % W
    left = (col != 0).astype(dtype)
    right = (col != W - 1).astype(dtype)
    return left, right


def _fill_shifted(buf, x, C, L, W):
    buf[...] = jnp.zeros(buf.shape, jnp.bfloat16)
    buf[0:C, pl.ds(W + 1, L - W)] = x[:, : L - W]
    buf[C:2 * C, pl.ds(1, L)] = x
    buf[2 * C:3 * C, pl.ds(0, L - W + 1)] = x[:, W - 1:]


def _conv_taps(buf, w_ref, L, W):
    ml, mr = _edge_masks(L, W, jnp.bfloat16)
    acc = jnp.dot(w_ref[0], buf[:, pl.ds(0, L)] * ml,
                  preferred_element_type=jnp.float32)
    acc = acc + jnp.dot(w_ref[1], buf[:, pl.ds(1, L)],
                        preferred_element_type=jnp.float32)
    acc = acc + jnp.dot(w_ref[2], buf[:, pl.ds(2, L)] * mr,
                        preferred_element_type=jnp.float32)
    return acc


def _conv_stats_call(xf, w, *, N, H, W, Cin, Cout):
    L = H * W

    def body(x_ref, w_ref, o_ref, s1_ref, s2_ref, buf):
        _fill_shifted(buf, x_ref[0], Cin, L, W)
        acc = _conv_taps(buf, w_ref, L, W)
        s1_ref[0] = jnp.sum(acc, axis=1, keepdims=True)
        s2_ref[0] = jnp.sum(acc * acc, axis=1, keepdims=True)
        o_ref[0] = acc.astype(jnp.bfloat16)

    return pl.pallas_call(
        body,
        out_shape=(
            jax.ShapeDtypeStruct((N, Cout, L), jnp.bfloat16),
            jax.ShapeDtypeStruct((N, Cout, 1), jnp.float32),
            jax.ShapeDtypeStruct((N, Cout, 1), jnp.float32),
        ),
        grid=(N,),
        in_specs=[
            pl.BlockSpec((1, Cin, L), lambda n: (n, 0, 0)),
            pl.BlockSpec((3, Cout, 3 * Cin), lambda n: (0, 0, 0)),
        ],
        out_specs=(
            pl.BlockSpec((1, Cout, L), lambda n: (n, 0, 0)),
            pl.BlockSpec((1, Cout, 1), lambda n: (n, 0, 0)),
            pl.BlockSpec((1, Cout, 1), lambda n: (n, 0, 0)),
        ),
        scratch_shapes=[pltpu.VMEM((3 * Cin, L + 2), jnp.bfloat16)],
        compiler_params=pltpu.CompilerParams(
            dimension_semantics=("parallel",),
            vmem_limit_bytes=_VMEM,
        ),
    )(xf, w)


def _finalize(s1_ref, s2_ref, g_ref, b_ref, m):
    mean = jnp.sum(s1_ref[...], axis=0) / m
    var = jnp.maximum(jnp.sum(s2_ref[...], axis=0) / m - mean * mean, 0.0)
    scale = g_ref[...] * jax.lax.rsqrt(var + _EPS)
    shift = b_ref[...] - mean * scale
    return scale, shift


def _bn_conv_stats_call(c1, w, s1, s2, g, b, *, N, H, W, Cin, Cout):
    L = H * W
    m = float(N * L)

    def body(x_ref, w_ref, s1_ref, s2_ref, g_ref, b_ref,
             o_ref, t1_ref, t2_ref, buf):
        scale, shift = _finalize(s1_ref, s2_ref, g_ref, b_ref, m)
        y = jnp.maximum(x_ref[0].astype(jnp.float32) * scale + shift, 0.0)
        _fill_shifted(buf, y.astype(jnp.bfloat16), Cin, L, W)
        acc = _conv_taps(buf, w_ref, L, W)
        t1_ref[0] = jnp.sum(acc, axis=1, keepdims=True)
        t2_ref[0] = jnp.sum(acc * acc, axis=1, keepdims=True)
        o_ref[0] = acc.astype(jnp.bfloat16)

    return pl.pallas_call(
        body,
        out_shape=(
            jax.ShapeDtypeStruct((N, Cout, L), jnp.bfloat16),
            jax.ShapeDtypeStruct((N, Cout, 1), jnp.float32),
            jax.ShapeDtypeStruct((N, Cout, 1), jnp.float32),
        ),
        grid=(N,),
        in_specs=[
            pl.BlockSpec((1, Cin, L), lambda n: (n, 0, 0)),
            pl.BlockSpec((3, Cout, 3 * Cin), lambda n: (0, 0, 0)),
            pl.BlockSpec((N, Cin, 1), lambda n: (0, 0, 0)),
            pl.BlockSpec((N, Cin, 1), lambda n: (0, 0, 0)),
            pl.BlockSpec((Cin, 1), lambda n: (0, 0)),
            pl.BlockSpec((Cin, 1), lambda n: (0, 0)),
        ],
        out_specs=(
            pl.BlockSpec((1, Cout, L), lambda n: (n, 0, 0)),
            pl.BlockSpec((1, Cout, 1), lambda n: (n, 0, 0)),
            pl.BlockSpec((1, Cout, 1), lambda n: (n, 0, 0)),
        ),
        scratch_shapes=[pltpu.VMEM((3 * Cin, L + 2), jnp.bfloat16)],
        compiler_params=pltpu.CompilerParams(
            dimension_semantics=("parallel",),
            vmem_limit_bytes=_VMEM,
        ),
    )(c1, w, s1, s2, g, b)


def _bn_relu_out_call(c2, s1, s2, g, b, *, N, H, W, Cout):
    L = H * W
    m = float(N * L)

    def body(x_ref, s1_ref, s2_ref, g_ref, b_ref, o_ref):
        scale, shift = _finalize(s1_ref, s2_ref, g_ref, b_ref, m)
        o_ref[0] = jnp.maximum(x_ref[0].astype(jnp.float32) * scale + shift, 0.0)

    return pl.pallas_call(
        body,
        out_shape=jax.ShapeDtypeStruct((N, Cout, L), jnp.float32),
        grid=(N,),
        in_specs=[
            pl.BlockSpec((1, Cout, L), lambda n: (n, 0, 0)),
            pl.BlockSpec((N, Cout, 1), lambda n: (0, 0, 0)),
            pl.BlockSpec((N, Cout, 1), lambda n: (0, 0, 0)),
            pl.BlockSpec((Cout, 1), lambda n: (0, 0)),
            pl.BlockSpec((Cout, 1), lambda n: (0, 0)),
        ],
        out_specs=pl.BlockSpec((1, Cout, L), lambda n: (n, 0, 0)),
        compiler_params=pltpu.CompilerParams(
            dimension_semantics=("parallel",),
            vmem_limit_bytes=_VMEM,
        ),
    )(c2, s1, s2, g, b)


def _pack_w(w, Cin, Cout):
    return (w.reshape(3, 3, Cin, Cout)
            .transpose(1, 3, 0, 2)
            .reshape(3, Cout, 3 * Cin)
            .astype(jnp.bfloat16))


def kernel(x, w1, g1, b1, w2, g2, b2):
    N, C0, H0, W0 = x.shape
    H, W = H0 // 2, W0 // 2
    L = H * W
    C1 = w1.shape[2]
    C2 = w2.shape[2]

    pooled = jnp.max(x.reshape(N, C0, H, 2, W, 2), axis=(3, 5))
    xf = pooled.astype(jnp.bfloat16).reshape(N, C0, L)

    w1p = _pack_w(w1, C0, C1)
    w2p = _pack_w(w2, C1, C2)
    g1r, b1r = g1.reshape(C1, 1), b1.reshape(C1, 1)
    g2r, b2r = g2.reshape(C2, 1), b2.reshape(C2, 1)

    c1, s1, s2 = _conv_stats_call(xf, w1p, N=N, H=H, W=W, Cin=C0, Cout=C1)
    c2, t1, t2 = _bn_conv_stats_call(c1, w2p, s1, s2, g1r, b1r,
                                     N=N, H=H, W=W, Cin=C1, Cout=C2)
    y = _bn_relu_out_call(c2, t1, t2, g2r, b2r, N=N, H=H, W=W, Cout=C2)
    return y.reshape(N, C2, H, W)
